# baseline (speedup 1.0000x reference)
import functools

import jax
import jax.numpy as jnp
from jax import lax
from jax.experimental import pallas as pl
from jax.experimental.pallas import tpu as pltpu
from jax.experimental.pallas import tpu_sc as plsc

NC = 2
NS = 16
L = 16
NW = NC * NS

BATCH = 16384
K = 16
B_PER_W = BATCH // NW
CHUNK = 128
N_CHUNK = B_PER_W // CHUNK
N_BLOCK = B_PER_W // L


def _mf_kernel(iu_hbm, ip_hbm, uemb_hbm, pemb_hbm, w_hbm, b_hbm, out_hbm,
               idxu_v, idxp_v, rows_u, rows_p, out_v, w_v, b_v, sem):
    wid = lax.axis_index("s") * NC + lax.axis_index("c")
    base_chunk = wid * N_CHUNK
    out_base = wid * B_PER_W

    pltpu.sync_copy(iu_hbm.at[pl.ds(base_chunk, N_CHUNK)], idxu_v)
    pltpu.sync_copy(ip_hbm.at[pl.ds(base_chunk, N_CHUNK)], idxp_v)
    pltpu.sync_copy(w_hbm, w_v)
    pltpu.sync_copy(b_hbm, b_v)

    copies = []
    for c in range(N_CHUNK):
        copies.append(pltpu.async_copy(
            uemb_hbm.at[idxu_v.at[c]], rows_u.at[pl.ds(c * CHUNK, CHUNK)],
            sem))
        copies.append(pltpu.async_copy(
            pemb_hbm.at[idxp_v.at[c]], rows_p.at[pl.ds(c * CHUNK, CHUNK)],
            sem))
    for cp in copies:
        cp.wait()

    iota = lax.iota(jnp.int32, L)
    col_ids = [jnp.full((L,), k, jnp.int32) for k in range(K)]
    wk_vecs = [w_v[k, :] for k in range(K)]
    bias = b_v[...]

    def block(blk, _):
        row_idx = blk * L + iota
        acc = bias
        for k in range(K):
            u = plsc.load_gather(rows_u, [row_idx, col_ids[k]])
            p = plsc.load_gather(rows_p, [row_idx, col_ids[k]])
            acc = acc + (u * p) * wk_vecs[k]
        out_v[pl.ds(blk * L, L)] = acc
        return 0

    lax.fori_loop(0, N_BLOCK, block, 0)

    pltpu.sync_copy(out_v, out_hbm.at[pl.ds(out_base, B_PER_W)])


@jax.jit
def _mf(iu, ip, user_emb, prob_emb, w_flat, b_vec):
    run = pl.kernel(
        _mf_kernel,
        out_type=jax.ShapeDtypeStruct((BATCH,), jnp.float32),
        mesh=plsc.VectorSubcoreMesh(core_axis_name="c", subcore_axis_name="s",
                                    num_cores=NC, num_subcores=NS),
        compiler_params=pltpu.CompilerParams(needs_layout_passes=False,
                                             use_tc_tiling_on_sc=False),
        scratch_types=[
            pltpu.VMEM((N_CHUNK, CHUNK), jnp.int32),
            pltpu.VMEM((N_CHUNK, CHUNK), jnp.int32),
            pltpu.VMEM((B_PER_W, K), jnp.float32),
            pltpu.VMEM((B_PER_W, K), jnp.float32),
            pltpu.VMEM((B_PER_W,), jnp.float32),
            pltpu.VMEM((K, L), jnp.float32),
            pltpu.VMEM((L,), jnp.float32),
            pltpu.SemaphoreType.DMA,
        ],
    )
    return run(iu, ip, user_emb, prob_emb, w_flat, b_vec)


def _tc_relayout(table):
    ti = lax.bitcast_convert_type(table, jnp.int32)
    ti = lax.optimization_barrier(ti + 5) - 5
    return lax.bitcast_convert_type(ti, jnp.float32)


def kernel(input_user, input_prob, user_emb, prob_emb, dense_w, dense_b):
    iu = input_user.reshape(NW * N_CHUNK, CHUNK)
    ip = input_prob.reshape(NW * N_CHUNK, CHUNK)
    w_bcast = jnp.broadcast_to(dense_w.reshape(K, 1), (K, L))
    b_vec = jnp.broadcast_to(dense_b, (L,))
    out = _mf(iu, ip, _tc_relayout(user_emb), _tc_relayout(prob_emb),
              w_bcast, b_vec)
    return out.reshape(BATCH, 1)


# speedup vs baseline: 1.6922x; 1.6922x over previous
import functools

import jax
import jax.numpy as jnp
from jax import lax
from jax.experimental import pallas as pl
from jax.experimental.pallas import tpu as pltpu
from jax.experimental.pallas import tpu_sc as plsc

NC = 2
NS = 16
L = 16
NW = NC * NS

BATCH = 16384
K = 16
HALF_V = 500000
B_PER_W = BATCH // NW
CHUNK = 128
N_CHUNK = B_PER_W // CHUNK
N_BLOCK = B_PER_W // L


def _mf_kernel(iu_hbm, ip_hbm, ua_hbm, ub_hbm, pemb_hbm, w_hbm, b_hbm,
               out_hbm, idxu_v, idxp_v, idxa_v, idxb_v, m_v, rows_ua,
               rows_ub, rows_p, out_v, w_v, b_v, sem):
    wid = lax.axis_index("s") * NC + lax.axis_index("c")
    base_chunk = wid * N_CHUNK
    out_base = wid * B_PER_W

    pltpu.sync_copy(iu_hbm.at[pl.ds(base_chunk, N_CHUNK)], idxu_v)
    pltpu.sync_copy(ip_hbm.at[pl.ds(base_chunk, N_CHUNK)], idxp_v)
    pltpu.sync_copy(w_hbm, w_v)
    pltpu.sync_copy(b_hbm, b_v)

    half = HALF_V
    for c in range(N_CHUNK):
        for j in range(CHUNK // L):
            v = idxu_v[c, pl.ds(j * L, L)]
            idxa_v[c, pl.ds(j * L, L)] = jnp.minimum(v, half - 1)
            idxb_v[c, pl.ds(j * L, L)] = jnp.maximum(v - half, 0)
            m_v[pl.ds((c * (CHUNK // L) + j) * L, L)] = jnp.where(
                v < half, 1.0, 0.0)

    copies = []
    for c in range(N_CHUNK):
        copies.append(pltpu.async_copy(
            ua_hbm.at[idxa_v.at[c]],
            rows_ua.at[pl.ds(c * CHUNK, CHUNK)], sem))
        copies.append(pltpu.async_copy(
            ub_hbm.at[idxb_v.at[c]],
            rows_ub.at[pl.ds(c * CHUNK, CHUNK)], sem))
        copies.append(pltpu.async_copy(
            pemb_hbm.at[idxp_v.at[c]], rows_p.at[pl.ds(c * CHUNK, CHUNK)],
            sem))
    for cp in copies:
        cp.wait()

    iota = lax.iota(jnp.int32, L)
    col_ids = [jnp.full((L,), k, jnp.int32) for k in range(K)]
    wk_vecs = [w_v[k, :] for k in range(K)]
    bias = b_v[...]

    def block(blk, _):
        row_idx = blk * L + iota
        m = m_v[pl.ds(blk * L, L)]
        acc = bias
        for k in range(K):
            ua = plsc.load_gather(rows_ua, [row_idx, col_ids[k]])
            ub = plsc.load_gather(rows_ub, [row_idx, col_ids[k]])
            u = ub + m * (ua - ub)
            p = plsc.load_gather(rows_p, [row_idx, col_ids[k]])
            acc = acc + (u * p) * wk_vecs[k]
        out_v[pl.ds(blk * L, L)] = acc
        return 0

    lax.fori_loop(0, N_BLOCK, block, 0)

    pltpu.sync_copy(out_v, out_hbm.at[pl.ds(out_base, B_PER_W)])


@jax.jit
def _mf(iu, ip, ua, ub, prob_emb, w_flat, b_vec):
    run = pl.kernel(
        _mf_kernel,
        out_type=jax.ShapeDtypeStruct((BATCH,), jnp.float32),
        mesh=plsc.VectorSubcoreMesh(core_axis_name="c", subcore_axis_name="s",
                                    num_cores=NC, num_subcores=NS),
        compiler_params=pltpu.CompilerParams(needs_layout_passes=False,
                                             use_tc_tiling_on_sc=False),
        scratch_types=[
            pltpu.VMEM((N_CHUNK, CHUNK), jnp.int32),
            pltpu.VMEM((N_CHUNK, CHUNK), jnp.int32),
            pltpu.VMEM((N_CHUNK, CHUNK), jnp.int32),
            pltpu.VMEM((N_CHUNK, CHUNK), jnp.int32),
            pltpu.VMEM((B_PER_W,), jnp.float32),
            pltpu.VMEM((B_PER_W, K), jnp.float32),
            pltpu.VMEM((B_PER_W, K), jnp.float32),
            pltpu.VMEM((B_PER_W, K), jnp.float32),
            pltpu.VMEM((B_PER_W,), jnp.float32),
            pltpu.VMEM((K, L), jnp.float32),
            pltpu.VMEM((L,), jnp.float32),
            pltpu.SemaphoreType.DMA,
        ],
    )
    return run(iu, ip, ua, ub, prob_emb, w_flat, b_vec)


def kernel(input_user, input_prob, user_emb, prob_emb, dense_w, dense_b):
    iu = input_user.reshape(NW * N_CHUNK, CHUNK)
    ip = input_prob.reshape(NW * N_CHUNK, CHUNK)
    ua = user_emb[:HALF_V]
    ub = user_emb[HALF_V:]
    w_bcast = jnp.broadcast_to(dense_w.reshape(K, 1), (K, L))
    b_vec = jnp.broadcast_to(dense_b, (L,))
    out = _mf(iu, ip, ua, ub, prob_emb, w_bcast, b_vec)
    return out.reshape(BATCH, 1)


# speedup vs baseline: 2.4570x; 1.4520x over previous
import functools

import jax
import jax.numpy as jnp
from jax import lax
from jax.experimental import pallas as pl
from jax.experimental.pallas import tpu as pltpu
from jax.experimental.pallas import tpu_sc as plsc

NC = 2
NS = 16
L = 16
NW = NC * NS

BATCH = 16384
K = 16
B_PER_W = BATCH // NW
CHUNK = 128
N_CHUNK = B_PER_W // CHUNK
N_BLOCK = B_PER_W // L


def _mf_kernel(iu_hbm, ip_hbm, uemb_hbm, pemb_hbm, w_hbm, b_hbm, out_hbm,
               idxu_v, idxp_v, rows_u, rows_p, out_v, w_v, b_v, sem):
    wid = lax.axis_index("s") * NC + lax.axis_index("c")
    base_chunk = wid * N_CHUNK
    out_base = wid * B_PER_W

    pltpu.sync_copy(iu_hbm.at[pl.ds(base_chunk, N_CHUNK)], idxu_v)
    pltpu.sync_copy(ip_hbm.at[pl.ds(base_chunk, N_CHUNK)], idxp_v)
    pltpu.sync_copy(w_hbm, w_v)
    pltpu.sync_copy(b_hbm, b_v)

    copies = []
    for c in range(N_CHUNK):
        copies.append(pltpu.async_copy(
            uemb_hbm.at[idxu_v.at[c]], rows_u.at[pl.ds(c * CHUNK, CHUNK)],
            sem))
        copies.append(pltpu.async_copy(
            pemb_hbm.at[idxp_v.at[c]], rows_p.at[pl.ds(c * CHUNK, CHUNK)],
            sem))
    for cp in copies:
        cp.wait()

    iota = lax.iota(jnp.int32, L)
    col_ids = [jnp.full((L,), k, jnp.int32) for k in range(K)]
    wk_vecs = [w_v[k, :] for k in range(K)]
    bias = b_v[...]

    def block(blk, _):
        row_idx = blk * L + iota
        acc = bias
        for k in range(K):
            u = plsc.load_gather(rows_u, [row_idx, col_ids[k]])
            p = plsc.load_gather(rows_p, [row_idx, col_ids[k]])
            acc = acc + (u * p) * wk_vecs[k]
        out_v[pl.ds(blk * L, L)] = acc
        return 0

    lax.fori_loop(0, N_BLOCK, block, 0)

    pltpu.sync_copy(out_v, out_hbm.at[pl.ds(out_base, B_PER_W)])


@jax.jit
def _mf(iu, ip, user_emb, prob_emb, w_flat, b_vec):
    run = pl.kernel(
        _mf_kernel,
        out_type=jax.ShapeDtypeStruct((BATCH,), jnp.float32),
        mesh=plsc.VectorSubcoreMesh(core_axis_name="c", subcore_axis_name="s",
                                    num_cores=NC, num_subcores=NS),
        compiler_params=pltpu.CompilerParams(needs_layout_passes=False,
                                             use_tc_tiling_on_sc=False),
        scratch_types=[
            pltpu.VMEM((N_CHUNK, CHUNK), jnp.int32),
            pltpu.VMEM((N_CHUNK, CHUNK), jnp.int32),
            pltpu.VMEM((B_PER_W, K), jnp.float32),
            pltpu.VMEM((B_PER_W, K), jnp.float32),
            pltpu.VMEM((B_PER_W,), jnp.float32),
            pltpu.VMEM((K, L), jnp.float32),
            pltpu.VMEM((L,), jnp.float32),
            pltpu.SemaphoreType.DMA,
        ],
    )
    return run(iu, ip, user_emb, prob_emb, w_flat, b_vec)


def kernel(input_user, input_prob, user_emb, prob_emb, dense_w, dense_b):
    iu = input_user.reshape(NW * N_CHUNK, CHUNK)
    ip = input_prob.reshape(NW * N_CHUNK, CHUNK)
    w_bcast = jnp.broadcast_to(dense_w.reshape(K, 1), (K, L))
    b_vec = jnp.broadcast_to(dense_b, (L,))
    out = _mf(iu, ip, user_emb, prob_emb, w_bcast, b_vec)
    return out.reshape(BATCH, 1)
